# Initial kernel scaffold; baseline (speedup 1.0000x reference)
import jax
import jax.numpy as jnp
from jax.experimental import pallas as pl


def kernel(X, T, proxies):
    raise NotImplementedError("write your pallas kernel here")


# speedup vs baseline: 3.6749x; 3.6749x over previous
import functools

import jax
import jax.numpy as jnp
from jax.experimental import pallas as pl
from jax.experimental.pallas import tpu as pltpu

_NB_CLASSES = 512
_SZ_EMBED = 512
_SCALE = 3.0
_SHIFTS = 16
_BLOCK = 1024


def _row_normalize(x, scale):
    n = jnp.sqrt(jnp.sum(x * x, axis=-1, keepdims=True))
    return x * (scale / jnp.maximum(n, 1e-12))


def _nca_body(xa_ref, xb_ref, p_ref, ta_ref, tb_ref, t2b_ref, out_ref,
              pn_ref, g_ref, acc_ref, *, nblk, block, ncls):
    s = pl.program_id(0)

    @pl.when(s == 0)
    def _init():
        acc_ref[0] = 0.0
        acc_ref[1] = 0.0
        pn_ref[:, :] = _row_normalize(p_ref[:, :], _SCALE)

    cols = jax.lax.broadcasted_iota(jnp.int32, (block, ncls), 1)

    def _logits_lse(xn):
        g = jax.lax.dot_general(
            xn, pn_ref[:, :], (((1,), (1,)), ((), ())),
            preferred_element_type=jnp.float32)
        logits = jnp.minimum(2.0 * g - 2.0 * _SCALE * _SCALE, 0.0)
        m = jnp.max(logits, axis=1, keepdims=True)
        lse = m + jnp.log(jnp.sum(jnp.exp(logits - m), axis=1, keepdims=True))
        return logits, lse

    def _label_val(logits, lbl):
        return jnp.sum(jnp.where(cols == lbl, logits, 0.0), axis=1,
                       keepdims=True)

    @pl.when(s < nblk)
    def _pass1():
        xn = _row_normalize(xa_ref[:, :], _SCALE)
        logits, lse = _logits_lse(xn)
        lt = _label_val(logits, ta_ref[0, :, :])
        acc_ref[0] += jnp.sum(lse - lt)
        gval = jnp.clip((lt + 2.0 * _SCALE * _SCALE) /
                        (2.0 * _SCALE * _SCALE), 0.0, 1.0)
        g_ref[jax.lax.rem(s, 2)] = gval

        @pl.when(s == 0)
        def _pin():
            g_ref[2] = gval

    @pl.when(s > 0)
    def _pass2():
        xb = xb_ref[:, :]
        xs = jnp.concatenate([xb[_SHIFTS:, :], xa_ref[:_SHIFTS, :]], axis=0)
        gb = g_ref[jax.lax.rem(s - 1, 2)]
        nxt = jnp.where(s < nblk, jax.lax.rem(s, 2), 2)
        gb1 = g_ref[nxt]
        g2 = jnp.concatenate([gb[_SHIFTS:, :], gb1[:_SHIFTS, :]], axis=0)
        lam = jnp.clip((gb + 1.0 - g2) * 0.5, 0.0, 1.0)
        virt = lam * xb + (1.0 - lam) * xs
        vn = _row_normalize(virt, _SCALE)
        logits, lse = _logits_lse(vn)
        l1 = _label_val(logits, tb_ref[0, :, :])
        l2 = _label_val(logits, t2b_ref[0, :, :])
        acc_ref[1] += jnp.sum(lse - lam * l1 - (1.0 - lam) * l2)

    @pl.when(s == nblk)
    def _fin():
        out_ref[:, :] = jnp.full(
            (1, 1), (acc_ref[0] + acc_ref[1]) / (nblk * block), jnp.float32)


@functools.partial(jax.jit, static_argnames=("interpret",))
def kernel(X, T, proxies, interpret=False):
    n, e = X.shape
    ncls = proxies.shape[0]
    block = _BLOCK
    nblk = n // block

    T = T.astype(jnp.int32)
    t_col = T.reshape(nblk, block, 1)
    t2_col = jnp.roll(T, -_SHIFTS).reshape(nblk, block, 1)

    grid = (nblk + 1,)
    out = pl.pallas_call(
        functools.partial(_nca_body, nblk=nblk, block=block, ncls=ncls),
        grid=grid,
        in_specs=[
            pl.BlockSpec((block, e), lambda s: (jax.lax.rem(s, nblk), 0)),
            pl.BlockSpec((block, e), lambda s: (jnp.maximum(s - 1, 0), 0)),
            pl.BlockSpec((ncls, e), lambda s: (0, 0)),
            pl.BlockSpec((1, block, 1),
                         lambda s: (jax.lax.rem(s, nblk), 0, 0)),
            pl.BlockSpec((1, block, 1),
                         lambda s: (jnp.maximum(s - 1, 0), 0, 0)),
            pl.BlockSpec((1, block, 1),
                         lambda s: (jnp.maximum(s - 1, 0), 0, 0)),
        ],
        out_specs=pl.BlockSpec((1, 1), lambda s: (0, 0)),
        out_shape=jax.ShapeDtypeStruct((1, 1), jnp.float32),
        scratch_shapes=[
            pltpu.VMEM((ncls, e), jnp.float32),
            pltpu.VMEM((3, block, 1), jnp.float32),
            pltpu.SMEM((2,), jnp.float32),
        ],
        interpret=interpret,
    )(X, X, proxies, t_col, t_col, t2_col)
    return out[0, 0]


# speedup vs baseline: 3.6876x; 1.0034x over previous
import functools

import jax
import jax.numpy as jnp
from jax.experimental import pallas as pl
from jax.experimental.pallas import tpu as pltpu

_NB_CLASSES = 512
_SZ_EMBED = 512
_SCALE = 3.0
_SHIFTS = 16
_BLOCK = 1024


def _row_normalize(x, scale):
    n = jnp.sqrt(jnp.sum(x * x, axis=-1, keepdims=True))
    return x * (scale / jnp.maximum(n, 1e-12))


def _nca_body(xa_ref, xb_ref, p_ref, ta_ref, tb_ref, t2b_ref, out_ref,
              pn_ref, g_ref, acc_ref, *, nblk, block, ncls):
    s = pl.program_id(0)

    @pl.when(s == 0)
    def _init():
        acc_ref[0] = 0.0
        acc_ref[1] = 0.0
        pn_ref[:, :] = _row_normalize(p_ref[:, :], _SCALE).astype(jnp.bfloat16)

    cols = jax.lax.broadcasted_iota(jnp.int32, (block, ncls), 1)

    def _logits_lse(xn):
        g = jax.lax.dot_general(
            xn.astype(jnp.bfloat16), pn_ref[:, :], (((1,), (1,)), ((), ())),
            preferred_element_type=jnp.float32)
        logits = jnp.minimum(2.0 * g - 2.0 * _SCALE * _SCALE, 0.0)
        m = jnp.max(logits, axis=1, keepdims=True)
        lse = m + jnp.log(jnp.sum(jnp.exp(logits - m), axis=1, keepdims=True))
        return logits, lse

    def _label_val(logits, lbl):
        return jnp.sum(jnp.where(cols == lbl, logits, 0.0), axis=1,
                       keepdims=True)

    @pl.when(s < nblk)
    def _pass1():
        xn = _row_normalize(xa_ref[:, :], _SCALE)
        logits, lse = _logits_lse(xn)
        lt = _label_val(logits, ta_ref[0, :, :])
        acc_ref[0] += jnp.sum(lse - lt)
        gval = jnp.clip((lt + 2.0 * _SCALE * _SCALE) /
                        (2.0 * _SCALE * _SCALE), 0.0, 1.0)
        g_ref[jax.lax.rem(s, 2)] = gval

        @pl.when(s == 0)
        def _pin():
            g_ref[2] = gval

    @pl.when(s > 0)
    def _pass2():
        xb = xb_ref[:, :]
        xs = jnp.concatenate([xb[_SHIFTS:, :], xa_ref[:_SHIFTS, :]], axis=0)
        gb = g_ref[jax.lax.rem(s - 1, 2)]
        nxt = jnp.where(s < nblk, jax.lax.rem(s, 2), 2)
        gb1 = g_ref[nxt]
        g2 = jnp.concatenate([gb[_SHIFTS:, :], gb1[:_SHIFTS, :]], axis=0)
        lam = jnp.clip((gb + 1.0 - g2) * 0.5, 0.0, 1.0)
        virt = lam * xb + (1.0 - lam) * xs
        vn = _row_normalize(virt, _SCALE)
        logits, lse = _logits_lse(vn)
        l1 = _label_val(logits, tb_ref[0, :, :])
        l2 = _label_val(logits, t2b_ref[0, :, :])
        acc_ref[1] += jnp.sum(lse - lam * l1 - (1.0 - lam) * l2)

    @pl.when(s == nblk)
    def _fin():
        out_ref[:, :] = jnp.full(
            (1, 1), (acc_ref[0] + acc_ref[1]) / (nblk * block), jnp.float32)


@functools.partial(jax.jit, static_argnames=("interpret",))
def kernel(X, T, proxies, interpret=False):
    n, e = X.shape
    ncls = proxies.shape[0]
    block = _BLOCK
    nblk = n // block

    T = T.astype(jnp.int32)
    t_col = T.reshape(nblk, block, 1)
    t2_col = jnp.roll(T, -_SHIFTS).reshape(nblk, block, 1)

    grid = (nblk + 1,)
    out = pl.pallas_call(
        functools.partial(_nca_body, nblk=nblk, block=block, ncls=ncls),
        grid=grid,
        in_specs=[
            pl.BlockSpec((block, e), lambda s: (jax.lax.rem(s, nblk), 0)),
            pl.BlockSpec((block, e), lambda s: (jnp.maximum(s - 1, 0), 0)),
            pl.BlockSpec((ncls, e), lambda s: (0, 0)),
            pl.BlockSpec((1, block, 1),
                         lambda s: (jax.lax.rem(s, nblk), 0, 0)),
            pl.BlockSpec((1, block, 1),
                         lambda s: (jnp.maximum(s - 1, 0), 0, 0)),
            pl.BlockSpec((1, block, 1),
                         lambda s: (jnp.maximum(s - 1, 0), 0, 0)),
        ],
        out_specs=pl.BlockSpec((1, 1), lambda s: (0, 0)),
        out_shape=jax.ShapeDtypeStruct((1, 1), jnp.float32),
        scratch_shapes=[
            pltpu.VMEM((ncls, e), jnp.bfloat16),
            pltpu.VMEM((3, block, 1), jnp.float32),
            pltpu.SMEM((2,), jnp.float32),
        ],
        interpret=interpret,
    )(X, X, proxies, t_col, t_col, t2_col)
    return out[0, 0]


# speedup vs baseline: 3.9805x; 1.0794x over previous
import functools

import jax
import jax.numpy as jnp
from jax.experimental import pallas as pl
from jax.experimental.pallas import tpu as pltpu

_SCALE = 3.0
_SHIFTS = 16
_BLOCK = 1024
_Z = 2.0 * _SCALE * _SCALE


def _unit_rows(x):
    n = jnp.sqrt(jnp.sum(x * x, axis=-1, keepdims=True))
    return x / jnp.maximum(n, 1e-12)


def _nca_body(xa_ref, xb_ref, p_ref, ta_ref, tb_ref, t2b_ref, out_ref,
              pn_ref, ones_ref, g_ref, acc_ref, *, nblk, block, ncls):
    s = pl.program_id(0)

    @pl.when(s == 0)
    def _init():
        acc_ref[0] = 0.0
        acc_ref[1] = 0.0
        pn_ref[:, :] = _unit_rows(p_ref[:, :]).astype(jnp.bfloat16)
        ones_ref[:, :] = jnp.ones_like(ones_ref)

    cols = jax.lax.broadcasted_iota(jnp.int32, (block, ncls), 1)

    def _rowsum(a):
        return jax.lax.dot_general(
            a.astype(jnp.bfloat16), ones_ref[:, :], (((1,), (0,)), ((), ())),
            preferred_element_type=jnp.float32)[:, :1]

    def _z_lse(xb16, sqsum):
        inv = _Z / jnp.maximum(jnp.sqrt(sqsum), 1e-12)
        m = jax.lax.dot_general(
            xb16, pn_ref[:, :], (((1,), (1,)), ((), ())),
            preferred_element_type=jnp.float32)
        z = m * inv
        lse = jnp.log(_rowsum(jnp.exp(z)))
        return z, lse

    @pl.when(s < nblk)
    def _pass1():
        x = xa_ref[:, :]
        xb16 = x.astype(jnp.bfloat16)
        z, lse = _z_lse(xb16, _rowsum(xb16 * xb16))
        lt = _rowsum(jnp.where(cols == ta_ref[0, :, :], z, 0.0))
        acc_ref[0] += jnp.sum(lse - lt)
        gval = jnp.clip(lt / _Z, 0.0, 1.0)
        g_ref[jax.lax.rem(s, 2)] = gval

        @pl.when(s == 0)
        def _pin():
            g_ref[2] = gval

    @pl.when(s > 0)
    def _pass2():
        xb = xb_ref[:, :]
        xs = jnp.concatenate([xb[_SHIFTS:, :], xa_ref[:_SHIFTS, :]], axis=0)
        gb = g_ref[jax.lax.rem(s - 1, 2)]
        gb1 = g_ref[jnp.where(s < nblk, jax.lax.rem(s, 2), 2)]
        g2 = jnp.concatenate([gb[_SHIFTS:, :], gb1[:_SHIFTS, :]], axis=0)
        lam = jnp.clip((gb + 1.0 - g2) * 0.5, 0.0, 1.0)
        virt = lam * xb + (1.0 - lam) * xs
        vb16 = virt.astype(jnp.bfloat16)
        z, lse = _z_lse(vb16, _rowsum(vb16 * vb16))
        w = (jnp.where(cols == tb_ref[0, :, :], lam, 0.0) +
             jnp.where(cols == t2b_ref[0, :, :], 1.0 - lam, 0.0))
        lw = _rowsum(z * w)
        acc_ref[1] += jnp.sum(lse - lw)

    @pl.when(s == nblk)
    def _fin():
        out_ref[:, :] = jnp.full(
            (1, 1), (acc_ref[0] + acc_ref[1]) / (nblk * block), jnp.float32)


@functools.partial(jax.jit, static_argnames=("interpret",))
def kernel(X, T, proxies, interpret=False):
    n, e = X.shape
    ncls = proxies.shape[0]
    block = _BLOCK
    nblk = n // block

    T = T.astype(jnp.int32)
    t_col = T.reshape(nblk, block, 1)
    t2_col = jnp.roll(T, -_SHIFTS).reshape(nblk, block, 1)

    out = pl.pallas_call(
        functools.partial(_nca_body, nblk=nblk, block=block, ncls=ncls),
        grid=(nblk + 1,),
        in_specs=[
            pl.BlockSpec((block, e), lambda s: (jax.lax.rem(s, nblk), 0)),
            pl.BlockSpec((block, e), lambda s: (jnp.maximum(s - 1, 0), 0)),
            pl.BlockSpec((ncls, e), lambda s: (0, 0)),
            pl.BlockSpec((1, block, 1),
                         lambda s: (jax.lax.rem(s, nblk), 0, 0)),
            pl.BlockSpec((1, block, 1),
                         lambda s: (jnp.maximum(s - 1, 0), 0, 0)),
            pl.BlockSpec((1, block, 1),
                         lambda s: (jnp.maximum(s - 1, 0), 0, 0)),
        ],
        out_specs=pl.BlockSpec((1, 1), lambda s: (0, 0)),
        out_shape=jax.ShapeDtypeStruct((1, 1), jnp.float32),
        scratch_shapes=[
            pltpu.VMEM((ncls, e), jnp.bfloat16),
            pltpu.VMEM((e, 128), jnp.bfloat16),
            pltpu.VMEM((3, block, 1), jnp.float32),
            pltpu.SMEM((2,), jnp.float32),
        ],
        interpret=interpret,
    )(X, X, proxies, t_col, t_col, t2_col)
    return out[0, 0]


# speedup vs baseline: 4.0194x; 1.0098x over previous
import functools

import jax
import jax.numpy as jnp
from jax.experimental import pallas as pl
from jax.experimental.pallas import tpu as pltpu

_SCALE = 3.0
_SHIFTS = 16
_BLOCK = 2048
_Z = 2.0 * _SCALE * _SCALE


def _unit_rows(x):
    n = jnp.sqrt(jnp.sum(x * x, axis=-1, keepdims=True))
    return x / jnp.maximum(n, 1e-12)


def _nca_body(xa_ref, xb_ref, p_ref, ta_ref, tb_ref, t2b_ref, out_ref,
              pn_ref, ones_ref, g_ref, acc_ref, *, nblk, block, ncls):
    s = pl.program_id(0)

    @pl.when(s == 0)
    def _init():
        acc_ref[0] = 0.0
        acc_ref[1] = 0.0
        pn_ref[:, :] = _unit_rows(p_ref[:, :]).astype(jnp.bfloat16)
        ones_ref[:, :] = jnp.ones_like(ones_ref)

    cols = jax.lax.broadcasted_iota(jnp.int32, (block, ncls), 1)

    def _rowsum(a):
        return jax.lax.dot_general(
            a.astype(jnp.bfloat16), ones_ref[:, :], (((1,), (0,)), ((), ())),
            preferred_element_type=jnp.float32)[:, :1]

    def _z_lse(xb16, sqsum):
        inv = _Z / jnp.maximum(jnp.sqrt(sqsum), 1e-12)
        m = jax.lax.dot_general(
            xb16, pn_ref[:, :], (((1,), (1,)), ((), ())),
            preferred_element_type=jnp.float32)
        z = m * inv
        lse = jnp.log(_rowsum(jnp.exp(z)))
        return z, lse

    @pl.when(s < nblk)
    def _pass1():
        x = xa_ref[:, :]
        xb16 = x.astype(jnp.bfloat16)
        z, lse = _z_lse(xb16, _rowsum(xb16 * xb16))
        lt = _rowsum(jnp.where(cols == ta_ref[0, :, :], z, 0.0))
        acc_ref[0] += jnp.sum(lse - lt)
        gval = jnp.clip(lt / _Z, 0.0, 1.0)
        g_ref[jax.lax.rem(s, 2)] = gval

        @pl.when(s == 0)
        def _pin():
            g_ref[2] = gval

    @pl.when(s > 0)
    def _pass2():
        xb = xb_ref[:, :]
        xs = jnp.concatenate([xb[_SHIFTS:, :], xa_ref[:_SHIFTS, :]], axis=0)
        gb = g_ref[jax.lax.rem(s - 1, 2)]
        gb1 = g_ref[jnp.where(s < nblk, jax.lax.rem(s, 2), 2)]
        g2 = jnp.concatenate([gb[_SHIFTS:, :], gb1[:_SHIFTS, :]], axis=0)
        lam = jnp.clip((gb + 1.0 - g2) * 0.5, 0.0, 1.0)
        virt = lam * xb + (1.0 - lam) * xs
        vb16 = virt.astype(jnp.bfloat16)
        z, lse = _z_lse(vb16, _rowsum(vb16 * vb16))
        l1 = _rowsum(jnp.where(cols == tb_ref[0, :, :], z, 0.0))
        l2 = _rowsum(jnp.where(cols == t2b_ref[0, :, :], z, 0.0))
        acc_ref[1] += jnp.sum(lse - lam * l1 - (1.0 - lam) * l2)

    @pl.when(s == nblk)
    def _fin():
        out_ref[:, :] = jnp.full(
            (1, 1), (acc_ref[0] + acc_ref[1]) / (nblk * block), jnp.float32)


@functools.partial(jax.jit, static_argnames=("interpret",))
def kernel(X, T, proxies, interpret=False):
    n, e = X.shape
    ncls = proxies.shape[0]
    block = _BLOCK
    nblk = n // block

    T = T.astype(jnp.int32)
    t_col = T.reshape(nblk, block, 1)
    t2_col = jnp.roll(T, -_SHIFTS).reshape(nblk, block, 1)

    out = pl.pallas_call(
        functools.partial(_nca_body, nblk=nblk, block=block, ncls=ncls),
        grid=(nblk + 1,),
        in_specs=[
            pl.BlockSpec((block, e), lambda s: (jax.lax.rem(s, nblk), 0)),
            pl.BlockSpec((block, e), lambda s: (jnp.maximum(s - 1, 0), 0)),
            pl.BlockSpec((ncls, e), lambda s: (0, 0)),
            pl.BlockSpec((1, block, 1),
                         lambda s: (jax.lax.rem(s, nblk), 0, 0)),
            pl.BlockSpec((1, block, 1),
                         lambda s: (jnp.maximum(s - 1, 0), 0, 0)),
            pl.BlockSpec((1, block, 1),
                         lambda s: (jnp.maximum(s - 1, 0), 0, 0)),
        ],
        out_specs=pl.BlockSpec((1, 1), lambda s: (0, 0)),
        out_shape=jax.ShapeDtypeStruct((1, 1), jnp.float32),
        scratch_shapes=[
            pltpu.VMEM((ncls, e), jnp.bfloat16),
            pltpu.VMEM((e, 128), jnp.bfloat16),
            pltpu.VMEM((3, block, 1), jnp.float32),
            pltpu.SMEM((2,), jnp.float32),
        ],
        interpret=interpret,
    )(X, X, proxies, t_col, t_col, t2_col)
    return out[0, 0]
